# Initial kernel scaffold; baseline (speedup 1.0000x reference)
import jax
import jax.numpy as jnp
from jax.experimental import pallas as pl


def kernel(atom_hiddens, a_scope):
    raise NotImplementedError("write your pallas kernel here")


# speedup vs baseline: 4.2413x; 4.2413x over previous
import functools

import jax
import jax.numpy as jnp
from jax import lax
from jax.experimental import pallas as pl
from jax.experimental.pallas import tpu as pltpu
from jax.experimental.pallas import tpu_sc as plsc

N_TOKENS = 32768
D = 128
N_MOLS = 16
L = 16
NV = D // L
CH = 512


def _sc_body(ah, starts_hbm, sizes_hbm, out, buf, acc_v, t0, t1, scope_v,
             shared):
    c = lax.axis_index("c")
    s = lax.axis_index("s")

    pltpu.sync_copy(starts_hbm, scope_v.at[0, pl.ds(0, N_MOLS)])
    pltpu.sync_copy(sizes_hbm, scope_v.at[1, pl.ds(0, N_MOLS)])

    def _at(row, idx):
        return scope_v[row, pl.ds(idx, L)][0]

    m = c * 8 + s // 2
    h = s % 2
    start = _at(0, m)
    size = _at(1, m)
    half = size // 2
    a = start + h * half
    b = a + half + h * (size - 2 * half)
    a8 = (a // 8) * 8
    nch = (b - a8 + CH - 1) // CH

    def chunk_body(i, carry):
        off = a8 + i * CH
        offc = jnp.minimum(off, N_TOKENS - CH)
        pltpu.sync_copy(ah.at[pl.ds(offc, CH), :], buf)
        rel_lo = jnp.maximum(a, off) - offc
        rel_hi = jnp.minimum(b, off + CH) - offc

        def row_body(r, acc):
            return tuple(acc[k] + buf[r, pl.ds(k * L, L)] for k in range(NV))

        return lax.fori_loop(rel_lo, rel_hi, row_body, carry)

    zeros = tuple(jnp.zeros((L,), jnp.float32) for _ in range(NV))
    accs = lax.fori_loop(0, nch, chunk_body, zeros)
    for k in range(NV):
        acc_v[pl.ds(k * L, L)] = accs[k]

    pltpu.sync_copy(acc_v, shared.at[s])
    plsc.subcore_barrier()

    @pl.when(s < 8)
    def _():
        pltpu.sync_copy(shared.at[2 * s], t0)
        pltpu.sync_copy(shared.at[2 * s + 1], t1)
        mrow = c * 8 + s
        szvec = jnp.full((L,), _at(1, mrow)).astype(jnp.float32)
        for k in range(NV):
            sl = pl.ds(k * L, L)
            t0[sl] = (t0[sl] + t1[sl]) / szvec
        pltpu.sync_copy(t0, out.at[pl.ds(mrow * D, D)])


_seg_mean = functools.partial(
    pl.kernel,
    out_type=jax.ShapeDtypeStruct((N_MOLS * D,), jnp.float32),
    mesh=plsc.VectorSubcoreMesh(core_axis_name="c", subcore_axis_name="s"),
    scratch_types=[
        pltpu.VMEM((CH, D), jnp.float32),
        pltpu.VMEM((D,), jnp.float32),
        pltpu.VMEM((D,), jnp.float32),
        pltpu.VMEM((D,), jnp.float32),
        pltpu.VMEM((2, 2 * L), jnp.int32),
        pltpu.VMEM_SHARED((N_MOLS, D), jnp.float32),
    ],
)(_sc_body)


def kernel(atom_hiddens, a_scope):
    starts = a_scope[:, 0]
    sizes = a_scope[:, 1]
    return _seg_mean(atom_hiddens, starts, sizes).reshape(N_MOLS, D)


# speedup vs baseline: 4.7956x; 1.1307x over previous
import functools

import jax
import jax.numpy as jnp
from jax import lax
from jax.experimental import pallas as pl
from jax.experimental.pallas import tpu as pltpu
from jax.experimental.pallas import tpu_sc as plsc

N_TOKENS = 32768
D = 128
N_MOLS = 16
L = 16
NV = D // L
CH = 256


def _sc_body(ah, starts_hbm, sizes_hbm, out, buf, acc_v, t0, t1, scope_v,
             shared, sem0, sem1):
    c = lax.axis_index("c")
    s = lax.axis_index("s")

    pltpu.sync_copy(starts_hbm, scope_v.at[0, pl.ds(0, N_MOLS)])
    pltpu.sync_copy(sizes_hbm, scope_v.at[1, pl.ds(0, N_MOLS)])

    def _at(row, idx):
        return scope_v[row, pl.ds(idx, L)][0]

    m = c * 8 + s // 2
    h = s % 2
    start = _at(0, m)
    size = _at(1, m)
    half = size // 2
    a = start + h * half
    b = a + half + h * (size - 2 * half)
    a8 = (a // 8) * 8
    nch = (b - a8 + CH - 1) // CH

    def _offc(k):
        return jnp.minimum(a8 + k * CH, N_TOKENS - CH)

    def _dma_start(k, buf_ref, sem):
        @pl.when(k < nch)
        def _():
            pltpu.async_copy(ah.at[pl.ds(_offc(k), CH), :], buf_ref, sem)

    def _dma_wait(k, buf_ref, sem):
        @pl.when(k < nch)
        def _():
            pltpu.make_async_copy(ah.at[pl.ds(0, CH), :], buf_ref, sem).wait()

    def _accumulate(k, buf_ref, carry):
        off = a8 + k * CH
        rel_lo = jnp.maximum(a, off) - _offc(k)
        rel_hi = jnp.minimum(b, off + CH) - _offc(k)

        def row_body(r, acc):
            return tuple(acc[kk] + buf_ref[r, pl.ds(kk * L, L)]
                         for kk in range(NV))

        return lax.fori_loop(rel_lo, rel_hi, row_body, carry)

    _dma_start(0, buf.at[0], sem0)
    _dma_start(1, buf.at[1], sem1)

    def pair_body(j, carry):
        k0 = 2 * j
        _dma_wait(k0, buf.at[0], sem0)
        carry = _accumulate(k0, buf.at[0], carry)
        _dma_start(k0 + 2, buf.at[0], sem0)
        _dma_wait(k0 + 1, buf.at[1], sem1)
        carry = _accumulate(k0 + 1, buf.at[1], carry)
        _dma_start(k0 + 3, buf.at[1], sem1)
        return carry

    zeros = tuple(jnp.zeros((L,), jnp.float32) for _ in range(NV))
    accs = lax.fori_loop(0, (nch + 1) // 2, pair_body, zeros)
    for k in range(NV):
        acc_v[pl.ds(k * L, L)] = accs[k]

    pltpu.sync_copy(acc_v, shared.at[s])
    plsc.subcore_barrier()

    @pl.when(s < 8)
    def _():
        pltpu.sync_copy(shared.at[2 * s], t0)
        pltpu.sync_copy(shared.at[2 * s + 1], t1)
        mrow = c * 8 + s
        szvec = jnp.full((L,), _at(1, mrow)).astype(jnp.float32)
        for k in range(NV):
            sl = pl.ds(k * L, L)
            t0[sl] = (t0[sl] + t1[sl]) / szvec
        pltpu.sync_copy(t0, out.at[pl.ds(mrow * D, D)])


_seg_mean = functools.partial(
    pl.kernel,
    out_type=jax.ShapeDtypeStruct((N_MOLS * D,), jnp.float32),
    mesh=plsc.VectorSubcoreMesh(core_axis_name="c", subcore_axis_name="s"),
    scratch_types=[
        pltpu.VMEM((2, CH, D), jnp.float32),
        pltpu.VMEM((D,), jnp.float32),
        pltpu.VMEM((D,), jnp.float32),
        pltpu.VMEM((D,), jnp.float32),
        pltpu.VMEM((2, 2 * L), jnp.int32),
        pltpu.VMEM_SHARED((N_MOLS, D), jnp.float32),
        pltpu.SemaphoreType.DMA,
        pltpu.SemaphoreType.DMA,
    ],
)(_sc_body)


def kernel(atom_hiddens, a_scope):
    starts = a_scope[:, 0]
    sizes = a_scope[:, 1]
    return _seg_mean(atom_hiddens, starts, sizes).reshape(N_MOLS, D)


# speedup vs baseline: 4.8037x; 1.0017x over previous
import functools

import jax
import jax.numpy as jnp
from jax import lax
from jax.experimental import pallas as pl
from jax.experimental.pallas import tpu as pltpu
from jax.experimental.pallas import tpu_sc as plsc

N_TOKENS = 32768
D = 128
N_MOLS = 16
L = 16
NV = D // L
CH = 256


def _sc_body(ah, starts_hbm, sizes_hbm, out, buf, acc_v, t0, t1, scope_v,
             shared, sem0, sem1):
    c = lax.axis_index("c")
    s = lax.axis_index("s")

    pltpu.sync_copy(starts_hbm, scope_v.at[0, pl.ds(0, N_MOLS)])
    pltpu.sync_copy(sizes_hbm, scope_v.at[1, pl.ds(0, N_MOLS)])

    def _at(row, idx):
        return scope_v[row, pl.ds(idx, L)][0]

    m = c * 8 + s // 2
    h = s % 2
    start = _at(0, m)
    size = _at(1, m)
    half = size // 2
    a = start + h * half
    b = a + half + h * (size - 2 * half)
    a8 = (a // 8) * 8
    nch = (b - a8 + CH - 1) // CH

    def _offc(k):
        return jnp.minimum(a8 + k * CH, N_TOKENS - CH)

    def _dma_start(k, buf_ref, sem):
        @pl.when(k < nch)
        def _():
            pltpu.async_copy(ah.at[pl.ds(_offc(k), CH), :], buf_ref, sem)

    def _dma_wait(k, buf_ref, sem):
        @pl.when(k < nch)
        def _():
            pltpu.make_async_copy(ah.at[pl.ds(0, CH), :], buf_ref, sem).wait()

    def _accumulate(k, buf_ref, carry):
        off = a8 + k * CH
        rel_lo = jnp.maximum(a, off) - _offc(k)
        rel_hi = jnp.minimum(b, off + CH) - _offc(k)

        def row_body(r, acc):
            return tuple(acc[kk] + buf_ref[r, pl.ds(kk * L, L)]
                         for kk in range(NV))

        return plsc.parallel_loop(rel_lo, rel_hi, step=1, unroll=8,
                                  carry=carry)(row_body)

    _dma_start(0, buf.at[0], sem0)
    _dma_start(1, buf.at[1], sem1)

    def pair_body(j, carry):
        k0 = 2 * j
        _dma_wait(k0, buf.at[0], sem0)
        carry = _accumulate(k0, buf.at[0], carry)
        _dma_start(k0 + 2, buf.at[0], sem0)
        _dma_wait(k0 + 1, buf.at[1], sem1)
        carry = _accumulate(k0 + 1, buf.at[1], carry)
        _dma_start(k0 + 3, buf.at[1], sem1)
        return carry

    zeros = tuple(jnp.zeros((L,), jnp.float32) for _ in range(NV))
    accs = lax.fori_loop(0, (nch + 1) // 2, pair_body, zeros)
    for k in range(NV):
        acc_v[pl.ds(k * L, L)] = accs[k]

    pltpu.sync_copy(acc_v, shared.at[s])
    plsc.subcore_barrier()

    @pl.when(s < 8)
    def _():
        pltpu.sync_copy(shared.at[2 * s], t0)
        pltpu.sync_copy(shared.at[2 * s + 1], t1)
        mrow = c * 8 + s
        szvec = jnp.full((L,), _at(1, mrow)).astype(jnp.float32)
        for k in range(NV):
            sl = pl.ds(k * L, L)
            t0[sl] = (t0[sl] + t1[sl]) / szvec
        pltpu.sync_copy(t0, out.at[pl.ds(mrow * D, D)])


_seg_mean = functools.partial(
    pl.kernel,
    out_type=jax.ShapeDtypeStruct((N_MOLS * D,), jnp.float32),
    mesh=plsc.VectorSubcoreMesh(core_axis_name="c", subcore_axis_name="s"),
    scratch_types=[
        pltpu.VMEM((2, CH, D), jnp.float32),
        pltpu.VMEM((D,), jnp.float32),
        pltpu.VMEM((D,), jnp.float32),
        pltpu.VMEM((D,), jnp.float32),
        pltpu.VMEM((2, 2 * L), jnp.int32),
        pltpu.VMEM_SHARED((N_MOLS, D), jnp.float32),
        pltpu.SemaphoreType.DMA,
        pltpu.SemaphoreType.DMA,
    ],
)(_sc_body)


def kernel(atom_hiddens, a_scope):
    starts = a_scope[:, 0]
    sizes = a_scope[:, 1]
    return _seg_mean(atom_hiddens, starts, sizes).reshape(N_MOLS, D)


# speedup vs baseline: 5.0047x; 1.0418x over previous
import functools

import jax
import jax.numpy as jnp
from jax import lax
from jax.experimental import pallas as pl
from jax.experimental.pallas import tpu as pltpu
from jax.experimental.pallas import tpu_sc as plsc

N_TOKENS = 32768
D = 128
N_MOLS = 16
L = 16
NV = D // L
CH = 128
NBUF = 4


def _sc_body(ah, starts_hbm, sizes_hbm, out, buf, acc_v, t0, t1, scope_v,
             shared, *sems):
    c = lax.axis_index("c")
    s = lax.axis_index("s")

    pltpu.sync_copy(starts_hbm, scope_v.at[0, pl.ds(0, N_MOLS)])
    pltpu.sync_copy(sizes_hbm, scope_v.at[1, pl.ds(0, N_MOLS)])

    def _at(row, idx):
        return scope_v[row, pl.ds(idx, L)][0]

    m = c * 8 + s // 2
    h = s % 2
    start = _at(0, m)
    size = _at(1, m)
    half = size // 2
    a = start + h * half
    b = a + half + h * (size - 2 * half)
    a8 = (a // 8) * 8
    nch = (b - a8 + CH - 1) // CH

    def _offc(k):
        return jnp.minimum(a8 + k * CH, N_TOKENS - CH)

    def _dma_start(k, buf_ref, sem):
        @pl.when(k < nch)
        def _():
            pltpu.async_copy(ah.at[pl.ds(_offc(k), CH), :], buf_ref, sem)

    def _dma_wait(k, buf_ref, sem):
        @pl.when(k < nch)
        def _():
            pltpu.make_async_copy(ah.at[pl.ds(0, CH), :], buf_ref, sem).wait()

    def _accumulate(k, buf_ref, carry):
        off = a8 + k * CH
        rel_lo = jnp.maximum(a, off) - _offc(k)
        rel_hi = jnp.minimum(b, off + CH) - _offc(k)

        def row_body(r, acc):
            return tuple(acc[kk] + buf_ref[r, pl.ds(kk * L, L)]
                         for kk in range(NV))

        return plsc.parallel_loop(rel_lo, rel_hi, step=1, unroll=8,
                                  carry=carry)(row_body)

    for bslot in range(NBUF):
        _dma_start(bslot, buf.at[bslot], sems[bslot])

    def ring_body(j, carry):
        k0 = NBUF * j
        for bslot in range(NBUF):
            k = k0 + bslot
            _dma_wait(k, buf.at[bslot], sems[bslot])
            carry = _accumulate(k, buf.at[bslot], carry)
            _dma_start(k + NBUF, buf.at[bslot], sems[bslot])
        return carry

    zeros = tuple(jnp.zeros((L,), jnp.float32) for _ in range(NV))
    accs = lax.fori_loop(0, (nch + NBUF - 1) // NBUF, ring_body, zeros)
    for k in range(NV):
        acc_v[pl.ds(k * L, L)] = accs[k]

    pltpu.sync_copy(acc_v, shared.at[s])
    plsc.subcore_barrier()

    @pl.when(s < 8)
    def _():
        pltpu.sync_copy(shared.at[2 * s], t0)
        pltpu.sync_copy(shared.at[2 * s + 1], t1)
        mrow = c * 8 + s
        szvec = jnp.full((L,), _at(1, mrow)).astype(jnp.float32)
        for k in range(NV):
            sl = pl.ds(k * L, L)
            t0[sl] = (t0[sl] + t1[sl]) / szvec
        pltpu.sync_copy(t0, out.at[pl.ds(mrow * D, D)])


_seg_mean = functools.partial(
    pl.kernel,
    out_type=jax.ShapeDtypeStruct((N_MOLS * D,), jnp.float32),
    mesh=plsc.VectorSubcoreMesh(core_axis_name="c", subcore_axis_name="s"),
    scratch_types=[
        pltpu.VMEM((NBUF, CH, D), jnp.float32),
        pltpu.VMEM((D,), jnp.float32),
        pltpu.VMEM((D,), jnp.float32),
        pltpu.VMEM((D,), jnp.float32),
        pltpu.VMEM((2, 2 * L), jnp.int32),
        pltpu.VMEM_SHARED((N_MOLS, D), jnp.float32),
    ] + [pltpu.SemaphoreType.DMA] * NBUF,
)(_sc_body)


def kernel(atom_hiddens, a_scope):
    starts = a_scope[:, 0]
    sizes = a_scope[:, 1]
    return _seg_mean(atom_hiddens, starts, sizes).reshape(N_MOLS, D)
